# baseline (speedup 1.0000x reference)
import functools

import numpy as np
import jax
import jax.numpy as jnp
from jax import lax
from jax.experimental import pallas as pl
from jax.experimental.pallas import tpu as pltpu
from jax.experimental.pallas import tpu_sc as plsc

VOCAB = 100000
DIM = 512
MAX_LEN = 2048
BATCH = 32
SEQ = 2048
SCALE = float(np.sqrt(DIM))

_info = plsc.get_sparse_core_info()
_NC, _NS, _L = _info.num_cores, _info.num_subcores, _info.num_lanes
NW = _NC * _NS
S_PER_W = SEQ // NW
VECS = DIM // 16


def _sinusoidal_pe_np(max_len, d):
    pos = np.arange(max_len, dtype=np.float32)[:, None]
    div = np.exp(np.arange(0, d, 2, dtype=np.float32) * (-np.log(10000.0) / d))
    pe = np.zeros((max_len, d), dtype=np.float32)
    pe[:, 0::2] = np.sin(pos * div)
    pe[:, 1::2] = np.cos(pos * div)
    return pe


_PE = _sinusoidal_pe_np(MAX_LEN, DIM)[:SEQ]

_mesh = plsc.VectorSubcoreMesh(core_axis_name="c", subcore_axis_name="s")


@functools.partial(
    pl.kernel,
    out_type=jax.ShapeDtypeStruct((BATCH, SEQ, DIM), jnp.float32),
    mesh=_mesh,
    scratch_types=[
        pltpu.VMEM((BATCH, S_PER_W), jnp.int32),
        pltpu.VMEM((S_PER_W, DIM), jnp.float32),
        pltpu.VMEM((S_PER_W, DIM), jnp.float32),
        pltpu.VMEM((S_PER_W, DIM), jnp.float32),
        pltpu.SemaphoreType.DMA,
        pltpu.SemaphoreType.DMA,
        pltpu.SemaphoreType.DMA,
        pltpu.SemaphoreType.DMA,
    ],
)
def _emb_sc(xt_hbm, table_hbm, pe_hbm, out_hbm, idx_v, pe_v,
            buf0, buf1, gsem0, gsem1, ssem0, ssem1):
    wid = lax.axis_index("s") * _NC + lax.axis_index("c")
    s0 = wid * S_PER_W
    pltpu.sync_copy(xt_hbm.at[wid], idx_v)
    pltpu.sync_copy(pe_hbm.at[pl.ds(s0, S_PER_W), :], pe_v)

    NCHUNK = 8
    ROWS_C = S_PER_W // NCHUNK

    def process(b, buf, ssem):
        for k in range(NCHUNK):
            def row_body(r, c, buf=buf):
                for v in range(VECS):
                    sl = pl.ds(v * 16, 16)
                    buf[r, sl] = buf[r, sl] * SCALE + pe_v[r, sl]
                return c

            lax.fori_loop(k * ROWS_C, (k + 1) * ROWS_C, row_body, 0)
            pltpu.async_copy(
                buf.at[pl.ds(k * ROWS_C, ROWS_C), :],
                out_hbm.at[b, pl.ds(s0 + k * ROWS_C, ROWS_C), :], ssem)
        pltpu.make_async_copy(
            buf, out_hbm.at[b, pl.ds(s0, S_PER_W), :], ssem).wait()

    pltpu.async_copy(table_hbm.at[idx_v.at[0]], buf0, gsem0)

    def pair_body(i, carry):
        b = 2 * i
        pltpu.make_async_copy(table_hbm.at[idx_v.at[b]], buf0, gsem0).wait()
        pltpu.async_copy(table_hbm.at[idx_v.at[b + 1]], buf1, gsem1)
        process(b, buf0, ssem0)
        pltpu.make_async_copy(table_hbm.at[idx_v.at[b + 1]], buf1, gsem1).wait()

        @pl.when(i < BATCH // 2 - 1)
        def _():
            pltpu.async_copy(table_hbm.at[idx_v.at[b + 2]], buf0, gsem0)

        process(b + 1, buf1, ssem1)
        return carry

    lax.fori_loop(0, BATCH // 2, pair_body, 0)


def kernel(x, table):
    pe = jnp.asarray(_PE)
    xt = jnp.swapaxes(x.reshape(BATCH, NW, S_PER_W), 0, 1)
    return _emb_sc(xt, table, pe)


# speedup vs baseline: 1.7676x; 1.7676x over previous
import functools

import numpy as np
import jax
import jax.numpy as jnp
from jax import lax
from jax.experimental import pallas as pl
from jax.experimental.pallas import tpu as pltpu
from jax.experimental.pallas import tpu_sc as plsc

VOCAB = 100000
DIM = 512
MAX_LEN = 2048
BATCH = 32
SEQ = 2048
SCALE = float(np.sqrt(DIM))

_info = plsc.get_sparse_core_info()
_NC, _NS, _L = _info.num_cores, _info.num_subcores, _info.num_lanes
NW = _NC * _NS
S_PER_W = SEQ // NW
VECS = DIM // 16


def _sinusoidal_pe_np(max_len, d):
    pos = np.arange(max_len, dtype=np.float32)[:, None]
    div = np.exp(np.arange(0, d, 2, dtype=np.float32) * (-np.log(10000.0) / d))
    pe = np.zeros((max_len, d), dtype=np.float32)
    pe[:, 0::2] = np.sin(pos * div)
    pe[:, 1::2] = np.cos(pos * div)
    return pe


_PE = _sinusoidal_pe_np(MAX_LEN, DIM)[:SEQ]

_mesh = plsc.VectorSubcoreMesh(core_axis_name="c", subcore_axis_name="s")


@functools.partial(
    pl.kernel,
    out_type=jax.ShapeDtypeStruct((BATCH, SEQ, DIM), jnp.float32),
    mesh=_mesh,
    scratch_types=[
        pltpu.VMEM((BATCH, S_PER_W), jnp.int32),
        pltpu.VMEM((S_PER_W, DIM), jnp.float32),
        pltpu.VMEM((S_PER_W, DIM), jnp.float32),
        pltpu.VMEM((S_PER_W, DIM), jnp.float32),
        pltpu.SemaphoreType.DMA,
        pltpu.SemaphoreType.DMA,
        pltpu.SemaphoreType.DMA,
        pltpu.SemaphoreType.DMA,
    ],
)
def _emb_sc(xt_hbm, table_hbm, pe_hbm, out_hbm, idx_v, pe_v,
            buf0, buf1, gsem0, gsem1, ssem0, ssem1):
    wid = lax.axis_index("s") * _NC + lax.axis_index("c")
    s0 = wid * S_PER_W
    pltpu.sync_copy(xt_hbm.at[wid], idx_v)
    pltpu.sync_copy(pe_hbm.at[pl.ds(s0, S_PER_W), :], pe_v)

    NCHUNK = 4
    ROWS_C = S_PER_W // NCHUNK

    def process(b, buf, ssem):
        for k in range(NCHUNK):
            def row_body(r, c, buf=buf):
                for v in range(VECS):
                    sl = pl.ds(v * 16, 16)
                    buf[r, sl] = buf[r, sl] * SCALE + pe_v[r, sl]
                return c

            lax.fori_loop(k * ROWS_C, (k + 1) * ROWS_C, row_body, 0)
            pltpu.async_copy(
                buf.at[pl.ds(k * ROWS_C, ROWS_C), :],
                out_hbm.at[b, pl.ds(s0 + k * ROWS_C, ROWS_C), :], ssem)
        pltpu.make_async_copy(
            buf, out_hbm.at[b, pl.ds(s0, S_PER_W), :], ssem).wait()

    pltpu.async_copy(table_hbm.at[idx_v.at[0]], buf0, gsem0)

    def pair_body(i, carry):
        b = 2 * i
        pltpu.make_async_copy(table_hbm.at[idx_v.at[b]], buf0, gsem0).wait()
        pltpu.async_copy(table_hbm.at[idx_v.at[b + 1]], buf1, gsem1)
        process(b, buf0, ssem0)
        pltpu.make_async_copy(table_hbm.at[idx_v.at[b + 1]], buf1, gsem1).wait()

        @pl.when(i < BATCH // 2 - 1)
        def _():
            pltpu.async_copy(table_hbm.at[idx_v.at[b + 2]], buf0, gsem0)

        process(b + 1, buf1, ssem1)
        return carry

    lax.fori_loop(0, BATCH // 2, pair_body, 0)


def kernel(x, table):
    pe = jnp.asarray(_PE)
    xt = jnp.swapaxes(x.reshape(BATCH, NW, S_PER_W), 0, 1)
    return _emb_sc(xt, table, pe)
